# baseline (speedup 1.0000x reference)
import functools
import math

import jax
import jax.numpy as jnp
from jax.experimental import pallas as pl
from jax.experimental.pallas import tpu as pltpu

_NUM_UNITS = 1024
_K = 64


def _tables_kernel(sa_ref, ca_ref, sb_ref, cb_ref, *, k, num_units, n_hi,
                   chunks):
    half_pi = jnp.float32(math.pi / 2.0)
    neg_log_rate = jnp.float32(-2.0 * math.log(10000.0) / num_units)
    scale = jnp.float32(num_units**0.5)
    col_b = jax.lax.broadcasted_iota(jnp.int32, (k, num_units), 1)
    w_b = jnp.exp(col_b.astype(jnp.float32) * neg_log_rate)
    lo = jax.lax.broadcasted_iota(jnp.int32, (k, num_units), 0)
    parity = (col_b & 1).astype(jnp.float32)
    b = lo.astype(jnp.float32) * w_b + parity * half_pi
    sb_ref[...] = jnp.sin(b) * scale
    cb_ref[...] = jnp.sin(b + half_pi) * scale
    shape_a = (n_hi // chunks, chunks, num_units)
    col_a = jax.lax.broadcasted_iota(jnp.int32, shape_a, 2)
    w_a = jnp.exp(col_a.astype(jnp.float32) * neg_log_rate)
    hi = (jax.lax.broadcasted_iota(jnp.int32, shape_a, 0) * chunks +
          jax.lax.broadcasted_iota(jnp.int32, shape_a, 1))
    a = (hi * k).astype(jnp.float32) * w_a
    sa_ref[...] = jnp.sin(a)
    ca_ref[...] = jnp.sin(a + half_pi)


def _pe_tile_kernel(sa_ref, ca_ref, sb_ref, cb_ref, out_ref, *, n, t_tile, k,
                    num_units):
    t = pl.program_id(0)
    chunks = t_tile // k
    for j in range(chunks):
        a_s = sa_ref[0, pl.ds(j, 1), :]
        a_c = ca_ref[0, pl.ds(j, 1), :]
        val = a_s * cb_ref[...] + a_c * sb_ref[...]
        out_ref[:, j * k:(j + 1) * k, :] = jnp.broadcast_to(
            val[None], (n, k, num_units))

    @pl.when(t == 0)
    def _zero_row0():
        out_ref[:, 0:1, :] = jnp.zeros((n, 1, num_units), jnp.float32)


def kernel(inputs):
    n, t_total = inputs.shape
    num_units = _NUM_UNITS
    t_tile = 256
    k = _K
    n_hi = t_total // k
    chunks = t_tile // k

    sa, ca, sb, cb = pl.pallas_call(
        functools.partial(_tables_kernel, k=k, num_units=num_units, n_hi=n_hi,
                          chunks=chunks),
        out_shape=[
            jax.ShapeDtypeStruct((n_hi // chunks, chunks, num_units),
                                 jnp.float32),
            jax.ShapeDtypeStruct((n_hi // chunks, chunks, num_units),
                                 jnp.float32),
            jax.ShapeDtypeStruct((k, num_units), jnp.float32),
            jax.ShapeDtypeStruct((k, num_units), jnp.float32),
        ],
    )()

    grid = (t_total // t_tile,)
    out = pl.pallas_call(
        functools.partial(_pe_tile_kernel, n=n, t_tile=t_tile, k=k,
                          num_units=num_units),
        grid=grid,
        in_specs=[
            pl.BlockSpec((1, chunks, num_units), lambda t: (t, 0, 0)),
            pl.BlockSpec((1, chunks, num_units), lambda t: (t, 0, 0)),
            pl.BlockSpec((k, num_units), lambda t: (0, 0)),
            pl.BlockSpec((k, num_units), lambda t: (0, 0)),
        ],
        out_specs=pl.BlockSpec((n, t_tile, num_units), lambda t: (0, t, 0)),
        out_shape=jax.ShapeDtypeStruct((n, t_total, num_units), jnp.float32),
        compiler_params=pltpu.CompilerParams(
            dimension_semantics=("parallel",)),
    )(sa, ca, sb, cb)
    return out


# speedup vs baseline: 1.1423x; 1.1423x over previous
import functools
import math

import jax
import jax.numpy as jnp
from jax.experimental import pallas as pl
from jax.experimental.pallas import tpu as pltpu

_NUM_UNITS = 1024
_K = 64


def _pe_tile_kernel(out_ref, sa_ref, ca_ref, sb_ref, cb_ref, *, n, t_tile, k,
                    num_units, n_hi):
    t = pl.program_id(0)
    half_pi = jnp.float32(math.pi / 2.0)
    neg_log_rate = jnp.float32(-2.0 * math.log(10000.0) / num_units)
    scale = jnp.float32(num_units**0.5)

    @pl.when(t == 0)
    def _init_tables():
        col_b = jax.lax.broadcasted_iota(jnp.int32, (k, num_units), 1)
        w_b = jnp.exp(col_b.astype(jnp.float32) * neg_log_rate)
        lo = jax.lax.broadcasted_iota(jnp.int32, (k, num_units), 0)
        parity = (col_b & 1).astype(jnp.float32)
        b = lo.astype(jnp.float32) * w_b + parity * half_pi
        sb_ref[...] = jnp.sin(b) * scale
        cb_ref[...] = jnp.sin(b + half_pi) * scale
        col_a = jax.lax.broadcasted_iota(jnp.int32, (n_hi, num_units), 1)
        w_a = jnp.exp(col_a.astype(jnp.float32) * neg_log_rate)
        hi = jax.lax.broadcasted_iota(jnp.int32, (n_hi, num_units), 0)
        a = (hi * k).astype(jnp.float32) * w_a
        sa_ref[...] = jnp.sin(a)
        ca_ref[...] = jnp.sin(a + half_pi)

    chunks = t_tile // k
    for j in range(chunks):
        hi_idx = t * chunks + j
        a_s = sa_ref[pl.ds(hi_idx, 1), :]
        a_c = ca_ref[pl.ds(hi_idx, 1), :]
        val = a_s * cb_ref[...] + a_c * sb_ref[...]
        out_ref[:, j * k:(j + 1) * k, :] = jnp.broadcast_to(
            val[None], (n, k, num_units))

    @pl.when(t == 0)
    def _zero_row0():
        out_ref[:, 0:1, :] = jnp.zeros((n, 1, num_units), jnp.float32)


def kernel(inputs):
    n, t_total = inputs.shape
    num_units = _NUM_UNITS
    t_tile = 512
    k = _K
    n_hi = t_total // k
    grid = (t_total // t_tile,)
    out = pl.pallas_call(
        functools.partial(_pe_tile_kernel, n=n, t_tile=t_tile, k=k,
                          num_units=num_units, n_hi=n_hi),
        grid=grid,
        out_specs=pl.BlockSpec((n, t_tile, num_units), lambda t: (0, t, 0)),
        out_shape=jax.ShapeDtypeStruct((n, t_total, num_units), jnp.float32),
        scratch_shapes=[
            pltpu.VMEM((n_hi, num_units), jnp.float32),
            pltpu.VMEM((n_hi, num_units), jnp.float32),
            pltpu.VMEM((k, num_units), jnp.float32),
            pltpu.VMEM((k, num_units), jnp.float32),
        ],
    )()
    return out


# speedup vs baseline: 1.1775x; 1.0308x over previous
import functools
import math

import jax
import jax.numpy as jnp
from jax.experimental import pallas as pl
from jax.experimental.pallas import tpu as pltpu

_NUM_UNITS = 1024
_K = 64


def _pe_tile_kernel(out_ref, sa_ref, ca_ref, sb_ref, cb_ref, *, n, t_tile, k,
                    num_units, n_hi):
    t = pl.program_id(0)
    half_pi = jnp.float32(math.pi / 2.0)
    neg_log_rate = jnp.float32(-2.0 * math.log(10000.0) / num_units)
    scale = jnp.float32(num_units**0.5)

    @pl.when(t == 0)
    def _init_tables():
        col_b = jax.lax.broadcasted_iota(jnp.int32, (k, num_units), 1)
        w_b = jnp.exp(col_b.astype(jnp.float32) * neg_log_rate)
        lo = jax.lax.broadcasted_iota(jnp.int32, (k, num_units), 0)
        parity = (col_b & 1).astype(jnp.float32)
        b = lo.astype(jnp.float32) * w_b + parity * half_pi
        sb_ref[...] = jnp.sin(b) * scale
        cb_ref[...] = jnp.sin(b + half_pi) * scale
        col_a = jax.lax.broadcasted_iota(jnp.int32, (n_hi, num_units), 1)
        w_a = jnp.exp(col_a.astype(jnp.float32) * neg_log_rate)
        hi = jax.lax.broadcasted_iota(jnp.int32, (n_hi, num_units), 0)
        a = (hi * k).astype(jnp.float32) * w_a
        sa_ref[...] = jnp.sin(a)
        ca_ref[...] = jnp.sin(a + half_pi)

    chunks = t_tile // k
    for j in range(chunks):
        hi_idx = t * chunks + j
        a_s = sa_ref[pl.ds(hi_idx, 1), :]
        a_c = ca_ref[pl.ds(hi_idx, 1), :]
        val = a_s * cb_ref[...] + a_c * sb_ref[...]
        out_ref[:, j * k:(j + 1) * k, :] = jnp.broadcast_to(
            val[None], (n, k, num_units))

    @pl.when(t == 0)
    def _zero_row0():
        out_ref[:, 0:1, :] = jnp.zeros((n, 1, num_units), jnp.float32)


def kernel(inputs):
    n, t_total = inputs.shape
    num_units = _NUM_UNITS
    t_tile = 256
    k = _K
    n_hi = t_total // k
    grid = (t_total // t_tile,)
    out = pl.pallas_call(
        functools.partial(_pe_tile_kernel, n=n, t_tile=t_tile, k=k,
                          num_units=num_units, n_hi=n_hi),
        grid=grid,
        out_specs=pl.BlockSpec((n, t_tile, num_units), lambda t: (0, t, 0)),
        out_shape=jax.ShapeDtypeStruct((n, t_total, num_units), jnp.float32),
        scratch_shapes=[
            pltpu.VMEM((n_hi, num_units), jnp.float32),
            pltpu.VMEM((n_hi, num_units), jnp.float32),
            pltpu.VMEM((k, num_units), jnp.float32),
            pltpu.VMEM((k, num_units), jnp.float32),
        ],
    )()
    return out


# speedup vs baseline: 1.3225x; 1.1232x over previous
import functools
import math

import jax
import jax.numpy as jnp
from jax.experimental import pallas as pl
from jax.experimental.pallas import tpu as pltpu

_NUM_UNITS = 1024
_K = 64

_TWO_PI_HI = 6.2831854820251465
_TWO_PI_LO = 2.0 * math.pi - 6.2831854820251465
_INV_TWO_PI = 1.0 / (2.0 * math.pi)
_S1 = 9.9998170357e-01
_S3 = -1.6662794909e-01
_S5 = 8.3104314374e-03
_S7 = -1.9286378239e-04
_S9 = 2.1582785572e-06


def _fast_sin(x):
    k = jnp.round(x * _INV_TWO_PI)
    r = (x - k * _TWO_PI_HI) - k * _TWO_PI_LO
    r2 = r * r
    return r * (_S1 + r2 * (_S3 + r2 * (_S5 + r2 * (_S7 + r2 * _S9))))


def _pe_tile_kernel(out_ref, sa_ref, ca_ref, sb_ref, cb_ref, *, n, t_tile, k,
                    num_units, n_hi):
    t = pl.program_id(0)
    half_pi = jnp.float32(math.pi / 2.0)
    neg_log_rate = jnp.float32(-2.0 * math.log(10000.0) / num_units)
    scale = jnp.float32(num_units**0.5)

    @pl.when(t == 0)
    def _init_tables():
        col_b = jax.lax.broadcasted_iota(jnp.int32, (k, num_units), 1)
        w_b = jnp.exp(col_b.astype(jnp.float32) * neg_log_rate)
        lo = jax.lax.broadcasted_iota(jnp.int32, (k, num_units), 0)
        parity = (col_b & 1).astype(jnp.float32)
        b = lo.astype(jnp.float32) * w_b + parity * half_pi
        sb_ref[...] = _fast_sin(b) * scale
        cb_ref[...] = _fast_sin(b + half_pi) * scale
        col_a = jax.lax.broadcasted_iota(jnp.int32, (n_hi, num_units), 1)
        w_a = jnp.exp(col_a.astype(jnp.float32) * neg_log_rate)
        hi = jax.lax.broadcasted_iota(jnp.int32, (n_hi, num_units), 0)
        a = (hi * k).astype(jnp.float32) * w_a
        sa_ref[...] = _fast_sin(a)
        ca_ref[...] = _fast_sin(a + half_pi)

    chunks = t_tile // k
    for j in range(chunks):
        hi_idx = t * chunks + j
        a_s = sa_ref[pl.ds(hi_idx, 1), :]
        a_c = ca_ref[pl.ds(hi_idx, 1), :]
        val = a_s * cb_ref[...] + a_c * sb_ref[...]
        out_ref[:, j * k:(j + 1) * k, :] = jnp.broadcast_to(
            val[None], (n, k, num_units))

    @pl.when(t == 0)
    def _zero_row0():
        out_ref[:, 0:1, :] = jnp.zeros((n, 1, num_units), jnp.float32)


def kernel(inputs):
    n, t_total = inputs.shape
    num_units = _NUM_UNITS
    t_tile = 256
    k = _K
    n_hi = t_total // k
    grid = (t_total // t_tile,)
    out = pl.pallas_call(
        functools.partial(_pe_tile_kernel, n=n, t_tile=t_tile, k=k,
                          num_units=num_units, n_hi=n_hi),
        grid=grid,
        out_specs=pl.BlockSpec((n, t_tile, num_units), lambda t: (0, t, 0)),
        out_shape=jax.ShapeDtypeStruct((n, t_total, num_units), jnp.float32),
        scratch_shapes=[
            pltpu.VMEM((n_hi, num_units), jnp.float32),
            pltpu.VMEM((n_hi, num_units), jnp.float32),
            pltpu.VMEM((k, num_units), jnp.float32),
            pltpu.VMEM((k, num_units), jnp.float32),
        ],
    )()
    return out


# speedup vs baseline: 1.3237x; 1.0009x over previous
import functools
import math

import jax
import jax.numpy as jnp
from jax.experimental import pallas as pl
from jax.experimental.pallas import tpu as pltpu

_NUM_UNITS = 1024
_K = 64

_TWO_PI_HI = 6.2831854820251465
_TWO_PI_LO = 2.0 * math.pi - 6.2831854820251465
_INV_TWO_PI = 1.0 / (2.0 * math.pi)
_S1 = 9.9998170357e-01
_S3 = -1.6662794909e-01
_S5 = 8.3104314374e-03
_S7 = -1.9286378239e-04
_S9 = 2.1582785572e-06


def _fast_sin(x):
    k = jnp.round(x * _INV_TWO_PI)
    r = (x - k * _TWO_PI_HI) - k * _TWO_PI_LO
    r2 = r * r
    return r * (_S1 + r2 * (_S3 + r2 * (_S5 + r2 * (_S7 + r2 * _S9))))


def _pe_tile_kernel(out_ref, sa_ref, ca_ref, sb_ref, cb_ref, *, n, t_tile, k,
                    num_units, n_hi):
    t = pl.program_id(0)
    half_pi = jnp.float32(math.pi / 2.0)
    neg_log_rate = jnp.float32(-2.0 * math.log(10000.0) / num_units)
    scale = jnp.float32(num_units**0.5)

    @pl.when(t == 0)
    def _init_tables():
        col8 = jax.lax.broadcasted_iota(jnp.int32, (8, num_units), 1)
        w_row = jnp.exp(col8.astype(jnp.float32) * neg_log_rate)[0:1, :]
        parity8 = (col8 & 1).astype(jnp.float32)
        phase = (parity8 * half_pi)[0:1, :]
        lo = jax.lax.broadcasted_iota(jnp.int32, (k, num_units), 0)
        b = lo.astype(jnp.float32) * w_row + phase
        sb_ref[...] = _fast_sin(b) * scale
        cb_ref[...] = _fast_sin(b + half_pi) * scale
        hi = jax.lax.broadcasted_iota(jnp.int32, (n_hi, num_units), 0)
        a = (hi * k).astype(jnp.float32) * w_row
        sa_ref[...] = _fast_sin(a)
        ca_ref[...] = _fast_sin(a + half_pi)

    chunks = t_tile // k
    for j in range(chunks):
        hi_idx = t * chunks + j
        a_s = sa_ref[pl.ds(hi_idx, 1), :]
        a_c = ca_ref[pl.ds(hi_idx, 1), :]
        val = a_s * cb_ref[...] + a_c * sb_ref[...]
        out_ref[:, j * k:(j + 1) * k, :] = jnp.broadcast_to(
            val[None], (n, k, num_units))

    @pl.when(t == 0)
    def _zero_row0():
        out_ref[:, 0:1, :] = jnp.zeros((n, 1, num_units), jnp.float32)


def kernel(inputs):
    n, t_total = inputs.shape
    num_units = _NUM_UNITS
    t_tile = 256
    k = _K
    n_hi = t_total // k
    grid = (t_total // t_tile,)
    out = pl.pallas_call(
        functools.partial(_pe_tile_kernel, n=n, t_tile=t_tile, k=k,
                          num_units=num_units, n_hi=n_hi),
        grid=grid,
        out_specs=pl.BlockSpec((n, t_tile, num_units), lambda t: (0, t, 0)),
        out_shape=jax.ShapeDtypeStruct((n, t_total, num_units), jnp.float32),
        scratch_shapes=[
            pltpu.VMEM((n_hi, num_units), jnp.float32),
            pltpu.VMEM((n_hi, num_units), jnp.float32),
            pltpu.VMEM((k, num_units), jnp.float32),
            pltpu.VMEM((k, num_units), jnp.float32),
        ],
    )()
    return out
